# Initial kernel scaffold; baseline (speedup 1.0000x reference)
import jax
import jax.numpy as jnp
from jax.experimental import pallas as pl


def kernel(x, e_i_ts):
    raise NotImplementedError("write your pallas kernel here")


# speedup vs baseline: 1.0276x; 1.0276x over previous
import functools

import jax
import jax.numpy as jnp
from jax import lax
from jax.experimental import pallas as pl
from jax.experimental.pallas import tpu as pltpu
from jax.experimental.pallas import tpu_sc as plsc


def _tc_stage(x_r, e):
    B, C, HW = x_r.shape
    K = e.shape[1]
    inv_count = 1.0 / (B * C * HW)

    def body(x_ref, e_ref, idx_ref, loss_ref, acc_ref):
        i = pl.program_id(0)
        xb = x_ref[0]
        et = e_ref[...]
        x2 = jnp.sum(xb * xb, axis=0)
        e2 = jnp.sum(et * et, axis=0)
        xe = lax.dot_general(
            xb, et, (((0,), (0,)), ((), ())),
            preferred_element_type=jnp.float32)
        scores = (x2[:, None] - 2.0 * xe) + e2[None, :]
        mins = jnp.min(scores, axis=1)
        k_iota = lax.broadcasted_iota(jnp.int32, scores.shape, 1)
        idx = jnp.min(jnp.where(scores == mins[:, None], k_iota, K), axis=1)
        idx_ref[0, 0, :] = idx

        @pl.when(i == 0)
        def _():
            acc_ref[...] = jnp.zeros_like(acc_ref)

        acc_ref[...] += mins.reshape(acc_ref.shape)

        @pl.when(i == pl.num_programs(0) - 1)
        def _():
            loss_ref[0, 0] = jnp.sum(acc_ref[...]) * inv_count

    return pl.pallas_call(
        body,
        grid=(B,),
        in_specs=[
            pl.BlockSpec((1, C, HW), lambda i: (i, 0, 0)),
            pl.BlockSpec((C, K), lambda i: (0, 0)),
        ],
        out_specs=[
            pl.BlockSpec((1, 1, HW), lambda i: (i, 0, 0)),
            pl.BlockSpec(block_shape=(1, 1), index_map=lambda i: (0, 0),
                         memory_space=pltpu.SMEM),
        ],
        out_shape=[
            jax.ShapeDtypeStruct((B, 1, HW), jnp.int32),
            jax.ShapeDtypeStruct((1, 1), jnp.float32),
        ],
        scratch_shapes=[pltpu.VMEM((8, HW // 8), jnp.float32)],
        compiler_params=pltpu.CompilerParams(
            dimension_semantics=("arbitrary",)),
    )(x_r, e)


def _sc_gather(table, idx2d):
    K, C = table.shape
    R, CH = idx2d.shape
    info = plsc.get_sparse_core_info()
    NW = info.num_cores * info.num_subcores
    rows_per_w = R // NW

    mesh = plsc.VectorSubcoreMesh(core_axis_name="c", subcore_axis_name="s")

    @functools.partial(
        pl.kernel,
        mesh=mesh,
        out_type=jax.ShapeDtypeStruct((R * CH, C), jnp.float32),
        scratch_types=[
            pltpu.VMEM((rows_per_w, CH), jnp.int32),
            pltpu.VMEM((CH, C), jnp.float32),
            pltpu.SemaphoreType.DMA,
        ],
    )
    def k(table_hbm, idx_hbm, out_hbm, idx_v, rows_v, sem):
        wid = lax.axis_index("s") * info.num_cores + lax.axis_index("c")
        row0 = wid * rows_per_w
        pltpu.sync_copy(idx_hbm.at[pl.ds(row0, rows_per_w)], idx_v)
        for c in range(rows_per_w):
            pltpu.async_copy(table_hbm.at[idx_v.at[c]], rows_v, sem).wait()
            pltpu.sync_copy(rows_v, out_hbm.at[pl.ds((row0 + c) * CH, CH)])

    return k(table, idx2d)


def kernel(x, e_i_ts):
    B, C, H, W = x.shape
    HW = H * W
    x_r = x.reshape(B, C, HW)
    idx3, loss_arr = _tc_stage(x_r, e_i_ts)
    loss = loss_arr[0, 0]
    table = e_i_ts.T
    idx2d = idx3.reshape(-1, 128)
    q_flat = _sc_gather(table, idx2d)
    q = q_flat.reshape(B, H, W, C).transpose(0, 3, 1, 2)
    return (q, loss, loss, idx3.reshape(B, HW))
